# initial kernel scaffold (unmeasured)
import jax
import jax.numpy as jnp
from jax import lax
from jax.experimental import pallas as pl
from jax.experimental.pallas import tpu as pltpu

B = 8
NB = 512
BS = 16
H = 8
D = 128
NKEY = NB * BS
NEG = -1e30


def kernel(Q, K, V, bt, lens):
    Qs = Q.reshape(B, H, D)
    lens2 = lens.reshape(B, 1)

    def body(q_ref, k_ref, v_ref, bt_ref, lens_ref, out_ref,
             o_send, o_recv, s_send, s_recv,
             send_sem_o, recv_sem_o, send_sem_s, recv_sem_s):
        my_x = lax.axis_index("x")
        my_y = lax.axis_index("y")
        partner = (my_x, 1 - my_y)

        barrier_sem = pltpu.get_barrier_semaphore()
        pl.semaphore_signal(
            barrier_sem, inc=1,
            device_id=partner, device_id_type=pl.DeviceIdType.MESH,
        )
        pl.semaphore_wait(barrier_sem, 1)

        bt_v = bt_ref[...]
        lens_v = lens_ref[...]
        page_id = (
            lax.broadcasted_iota(jnp.int32, (B, NB, NB), 2) + my_y * NB
        )
        jidx = lax.broadcasted_iota(jnp.int32, (B, NB, NB), 1)
        hit = (bt_v[:, :, None] == page_id) & (jidx < lens_v[:, :, None])
        counts = jnp.sum(hit.astype(jnp.float32), axis=1)
        counts_keys = jnp.broadcast_to(
            counts[:, :, None], (B, NB, BS)
        ).reshape(B, NKEY)
        valid = counts_keys > 0.0

        q_v = q_ref[...]
        scale = D ** -0.5

        m_cols = []
        l_cols = []
        for h in range(H):
            kh = k_ref[:, :, h, :].reshape(NKEY, D)
            vh = v_ref[:, :, h, :].reshape(NKEY, D)
            qh = q_v[:, h, :]
            s = lax.dot_general(
                qh, kh, (((1,), (1,)), ((), ())),
                preferred_element_type=jnp.float32,
            ) * scale
            s = jnp.where(valid, s, NEG)
            m_h = jnp.max(s, axis=1, keepdims=True)
            e_h = jnp.exp(s - m_h) * counts_keys
            l_h = jnp.sum(e_h, axis=1, keepdims=True)
            o_h = lax.dot_general(
                e_h, vh, (((1,), (0,)), ((), ())),
                preferred_element_type=jnp.float32,
            )
            o_send[:, h, :] = o_h
            m_cols.append(m_h)
            l_cols.append(l_h)

        m_all = jnp.concatenate(m_cols, axis=1)
        l_all = jnp.concatenate(l_cols, axis=1)
        s_send[0] = m_all
        s_send[1] = l_all

        rdma_o = pltpu.make_async_remote_copy(
            src_ref=o_send, dst_ref=o_recv,
            send_sem=send_sem_o, recv_sem=recv_sem_o,
            device_id=partner, device_id_type=pl.DeviceIdType.MESH,
        )
        rdma_s = pltpu.make_async_remote_copy(
            src_ref=s_send, dst_ref=s_recv,
            send_sem=send_sem_s, recv_sem=recv_sem_s,
            device_id=partner, device_id_type=pl.DeviceIdType.MESH,
        )
        rdma_o.start()
        rdma_s.start()
        rdma_o.wait()
        rdma_s.wait()

        m_r = s_recv[0]
        l_r = s_recv[1]
        m_f = jnp.maximum(m_all, m_r)
        a_l = jnp.exp(m_all - m_f)
        a_r = jnp.exp(m_r - m_f)
        l_f = l_all * a_l + l_r * a_r
        for h in range(H):
            w_l = (a_l[:, h:h + 1] / l_f[:, h:h + 1])
            w_r = (a_r[:, h:h + 1] / l_f[:, h:h + 1])
            out_ref[:, h, :] = o_send[:, h, :] * w_l + o_recv[:, h, :] * w_r

    out = pl.pallas_call(
        body,
        out_shape=jax.ShapeDtypeStruct((B, H, D), jnp.float32),
        in_specs=[
            pl.BlockSpec(memory_space=pltpu.VMEM),
            pl.BlockSpec(memory_space=pltpu.VMEM),
            pl.BlockSpec(memory_space=pltpu.VMEM),
            pl.BlockSpec(memory_space=pltpu.VMEM),
            pl.BlockSpec(memory_space=pltpu.VMEM),
        ],
        out_specs=pl.BlockSpec(memory_space=pltpu.VMEM),
        scratch_shapes=[
            pltpu.VMEM((B, H, D), jnp.float32),
            pltpu.VMEM((B, H, D), jnp.float32),
            pltpu.VMEM((2, B, H), jnp.float32),
            pltpu.VMEM((2, B, H), jnp.float32),
            pltpu.SemaphoreType.DMA,
            pltpu.SemaphoreType.DMA,
            pltpu.SemaphoreType.DMA,
            pltpu.SemaphoreType.DMA,
        ],
        compiler_params=pltpu.CompilerParams(collective_id=0),
    )(Qs, K, V, bt, lens2)

    return out.reshape(B, 1, H, D)


# baseline (device time: 31443 ns/iter reference)
import jax
import jax.numpy as jnp
from jax import lax
from jax.experimental import pallas as pl
from jax.experimental.pallas import tpu as pltpu

B = 8
NB = 512
BS = 16
H = 8
D = 128
NKEY = NB * BS
NEG = -1e30


def kernel(Q, K, V, bt, lens):
    Qs = Q.reshape(B, H, D)
    lens2 = lens.reshape(B, 1)

    def body(q_ref, k_ref, v_ref, bt_ref, lens_ref, out_ref,
             k_buf, v_buf, o_send, o_recv, s_send, s_recv,
             k_sems, v_sems,
             send_sem_o, recv_sem_o, send_sem_s, recv_sem_s):
        my_x = lax.axis_index("x")
        my_y = lax.axis_index("y")
        partner = (my_x, 1 - my_y)

        barrier_sem = pltpu.get_barrier_semaphore()
        pl.semaphore_signal(
            barrier_sem, inc=1,
            device_id=partner, device_id_type=pl.DeviceIdType.MESH,
        )
        pl.semaphore_wait(barrier_sem, 1)

        def kv_dma(h, slot):
            return (
                pltpu.make_async_copy(
                    k_ref.at[:, :, h, :], k_buf.at[slot], k_sems.at[slot]
                ),
                pltpu.make_async_copy(
                    v_ref.at[:, :, h, :], v_buf.at[slot], v_sems.at[slot]
                ),
            )

        kd0, vd0 = kv_dma(0, 0)
        kd0.start()
        vd0.start()

        bt_v = bt_ref[...]
        lens_v = lens_ref[...]
        page_id = (
            lax.broadcasted_iota(jnp.int32, (B, NB, NB), 2) + my_y * NB
        )
        jidx = lax.broadcasted_iota(jnp.int32, (B, NB, NB), 1)
        hit = (bt_v[:, :, None] == page_id) & (jidx < lens_v[:, :, None])
        counts = jnp.sum(hit.astype(jnp.float32), axis=1)
        counts_keys = jnp.broadcast_to(
            counts[:, :, None], (B, NB, BS)
        ).reshape(B, NKEY)
        valid = counts_keys > 0.0

        q_v = q_ref[...]
        scale = D ** -0.5

        m_cols = []
        l_cols = []
        for h in range(H):
            slot = h % 2
            if h + 1 < H:
                kd, vd = kv_dma(h + 1, (h + 1) % 2)
                kd.start()
                vd.start()
            kw, vw = kv_dma(h, slot)
            kw.wait()
            vw.wait()

            kh = k_buf[slot].reshape(NKEY, D)
            vh = v_buf[slot].reshape(NKEY, D)
            qh = q_v[:, h, :]
            s = lax.dot_general(
                qh, kh, (((1,), (1,)), ((), ())),
                preferred_element_type=jnp.float32,
            ) * scale
            s = jnp.where(valid, s, NEG)
            m_h = jnp.max(s, axis=1, keepdims=True)
            e_h = jnp.exp(s - m_h) * counts_keys
            l_h = jnp.sum(e_h, axis=1, keepdims=True)
            o_h = lax.dot_general(
                e_h, vh, (((1,), (0,)), ((), ())),
                preferred_element_type=jnp.float32,
            )
            o_send[:, h, :] = o_h
            m_cols.append(m_h)
            l_cols.append(l_h)

        m_all = jnp.concatenate(m_cols, axis=1)
        l_all = jnp.concatenate(l_cols, axis=1)
        s_send[0] = m_all
        s_send[1] = l_all

        rdma_o = pltpu.make_async_remote_copy(
            src_ref=o_send, dst_ref=o_recv,
            send_sem=send_sem_o, recv_sem=recv_sem_o,
            device_id=partner, device_id_type=pl.DeviceIdType.MESH,
        )
        rdma_s = pltpu.make_async_remote_copy(
            src_ref=s_send, dst_ref=s_recv,
            send_sem=send_sem_s, recv_sem=recv_sem_s,
            device_id=partner, device_id_type=pl.DeviceIdType.MESH,
        )
        rdma_o.start()
        rdma_s.start()
        rdma_o.wait()
        rdma_s.wait()

        m_r = s_recv[0]
        l_r = s_recv[1]
        m_f = jnp.maximum(m_all, m_r)
        a_l = jnp.exp(m_all - m_f)
        a_r = jnp.exp(m_r - m_f)
        l_f = l_all * a_l + l_r * a_r
        for h in range(H):
            w_l = (a_l[:, h:h + 1] / l_f[:, h:h + 1])
            w_r = (a_r[:, h:h + 1] / l_f[:, h:h + 1])
            out_ref[:, h, :] = o_send[:, h, :] * w_l + o_recv[:, h, :] * w_r

    out = pl.pallas_call(
        body,
        out_shape=jax.ShapeDtypeStruct((B, H, D), jnp.float32),
        in_specs=[
            pl.BlockSpec(memory_space=pltpu.VMEM),
            pl.BlockSpec(memory_space=pl.ANY),
            pl.BlockSpec(memory_space=pl.ANY),
            pl.BlockSpec(memory_space=pltpu.VMEM),
            pl.BlockSpec(memory_space=pltpu.VMEM),
        ],
        out_specs=pl.BlockSpec(memory_space=pltpu.VMEM),
        scratch_shapes=[
            pltpu.VMEM((2, NB, BS, D), jnp.float32),
            pltpu.VMEM((2, NB, BS, D), jnp.float32),
            pltpu.VMEM((B, H, D), jnp.float32),
            pltpu.VMEM((B, H, D), jnp.float32),
            pltpu.VMEM((2, B, H), jnp.float32),
            pltpu.VMEM((2, B, H), jnp.float32),
            pltpu.SemaphoreType.DMA((2,)),
            pltpu.SemaphoreType.DMA((2,)),
            pltpu.SemaphoreType.DMA,
            pltpu.SemaphoreType.DMA,
            pltpu.SemaphoreType.DMA,
            pltpu.SemaphoreType.DMA,
        ],
        compiler_params=pltpu.CompilerParams(collective_id=0),
    )(Qs, K, V, bt, lens2)

    return out.reshape(B, 1, H, D)
